# baseline (speedup 1.0000x reference)
import functools

import jax
import jax.numpy as jnp
from jax.experimental import pallas as pl
from jax.experimental.pallas import tpu as pltpu
from jax.experimental.pallas import tpu_sc as plsc

TOP_K = 2
ROUTED_SCALING_FACTOR = 2.5
TM = 128

SC_NC = 2
SC_NS = 16
SC_NW = SC_NC * SC_NS
SC_LANES = 16


def _router_body(x_ref, gw_ref, bias_ref, idx_ref, w_ref):
    x = x_ref[...]
    gw = gw_ref[...]
    logits = jax.lax.dot_general(x, gw, (((1,), (1,)), ((), ())),
                                 preferred_element_type=jnp.float32)
    scores = jax.nn.sigmoid(logits)
    s = scores + bias_ref[...]
    t, e = s.shape
    iota = jax.lax.broadcasted_iota(jnp.int32, (t, e), 1)
    big = jnp.asarray(e, jnp.int32)
    m1 = jnp.max(s, axis=1, keepdims=True)
    i1 = jnp.min(jnp.where(s == m1, iota, big), axis=1, keepdims=True)
    oh1 = (iota == i1)
    s2 = jnp.where(oh1, -jnp.inf, s)
    m2 = jnp.max(s2, axis=1, keepdims=True)
    i2 = jnp.min(jnp.where(s2 == m2, iota, big), axis=1, keepdims=True)
    oh2 = (iota == i2)
    w1 = jnp.sum(jnp.where(oh1, scores, 0.0), axis=1, keepdims=True)
    w2 = jnp.sum(jnp.where(oh2, scores, 0.0), axis=1, keepdims=True)
    scale = ROUTED_SCALING_FACTOR / (w1 + w2 + 1e-20)
    idx_ref[...] = jnp.concatenate([i1, i2], axis=1)
    w_ref[...] = jnp.concatenate([w1 * scale, w2 * scale], axis=1)


def _router(x, gate_weight, bias):
    t, d = x.shape
    e = gate_weight.shape[0]
    return pl.pallas_call(
        _router_body,
        out_shape=(jax.ShapeDtypeStruct((t, TOP_K), jnp.int32),
                   jax.ShapeDtypeStruct((t, TOP_K), jnp.float32)),
    )(x, gate_weight, bias.reshape(1, e))


def _dispatch_body(ne, tm, idx_ref, w_ref,
                   rid_ref, rw_ref, teid_ref, pos_ref,
                   cnt_ref, off_ref, cur_ref):
    p = idx_ref.shape[0]
    pad_t = rid_ref.shape[0]
    nt = teid_ref.shape[0]

    for e in range(ne):
        cnt_ref[e] = 0

    def count(i, _):
        e = idx_ref[i]
        cnt_ref[e] = cnt_ref[e] + 1
        return 0
    jax.lax.fori_loop(0, p, count, 0)

    for e in range(ne):
        off_ref[e] = 0
    for e in range(ne):
        padded = ((cnt_ref[e] + tm - 1) // tm) * tm
        for e2 in range(e + 1, ne):
            off_ref[e2] = off_ref[e2] + padded
        cur_ref[e] = 0

    def teid(i, _):
        row = i * tm
        acc = 0
        for e in range(1, ne):
            acc = acc + jnp.where(off_ref[e] <= row, 1, 0)
        teid_ref[i] = jnp.minimum(acc, ne - 1)
        return 0
    jax.lax.fori_loop(0, nt, teid, 0)

    def clear(s, _):
        rid_ref[s] = 0
        rw_ref[s] = 0.0
        return 0
    jax.lax.fori_loop(0, pad_t, clear, 0)

    def scatter(i, _):
        e = idx_ref[i]
        slot = off_ref[e] + cur_ref[e]
        cur_ref[e] = cur_ref[e] + 1
        rid_ref[slot] = i // TOP_K
        rw_ref[slot] = w_ref[i]
        pos_ref[i] = slot
        return 0
    jax.lax.fori_loop(0, p, scatter, 0)


def _dispatch(idx, w, ne, pad_t, nt, tm):
    p = idx.shape[0] * TOP_K
    idx = idx.reshape(p)
    w = w.reshape(p)
    smem = functools.partial(pl.BlockSpec, memory_space=pltpu.SMEM)
    return pl.pallas_call(
        functools.partial(_dispatch_body, ne, tm),
        in_specs=[smem(), smem()],
        out_specs=(smem(), smem(), smem(), smem()),
        out_shape=(jax.ShapeDtypeStruct((pad_t,), jnp.int32),
                   jax.ShapeDtypeStruct((pad_t,), jnp.float32),
                   jax.ShapeDtypeStruct((nt,), jnp.int32),
                   jax.ShapeDtypeStruct((p,), jnp.int32)),
        scratch_shapes=[pltpu.SMEM((ne,), jnp.int32),
                        pltpu.SMEM((ne,), jnp.int32),
                        pltpu.SMEM((ne,), jnp.int32)],
    )(idx, w)


def _gather_body(x_ref, rid2_ref, xs_ref):
    t = x_ref.shape[0]
    tm = xs_ref.shape[0]
    onehot = (rid2_ref[...] == jax.lax.broadcasted_iota(jnp.int32, (tm, t), 1)
              ).astype(jnp.float32)
    xs_ref[...] = jax.lax.dot_general(onehot, x_ref[...],
                                      (((1,), (0,)), ((), ())),
                                      preferred_element_type=jnp.float32)


def _gather_tc(x, rid2, pad_t, tm):
    t, d = x.shape
    return pl.pallas_call(
        _gather_body,
        grid=(pad_t // tm,),
        in_specs=[pl.BlockSpec((t, d), lambda i: (0, 0)),
                  pl.BlockSpec((tm, 1), lambda i: (i, 0))],
        out_specs=pl.BlockSpec((tm, d), lambda i: (i, 0)),
        out_shape=jax.ShapeDtypeStruct((pad_t, d), jnp.float32),
    )(x, rid2)


def _gather_sc(x, rid, pad_t):
    t, d = x.shape
    rows_per_w = pad_t // SC_NW
    ch = 32
    chunks = rows_per_w // ch
    mesh = plsc.VectorSubcoreMesh(core_axis_name="c", subcore_axis_name="s",
                                  num_cores=SC_NC, num_subcores=SC_NS)

    @functools.partial(
        pl.kernel, mesh=mesh,
        out_type=jax.ShapeDtypeStruct((pad_t, d), jnp.float32),
        scratch_types=[pltpu.VMEM((ch,), jnp.int32),
                       pltpu.VMEM((ch, d), jnp.float32),
                       pltpu.SemaphoreType.DMA],
    )
    def gather_k(x_hbm, rid_hbm, out_hbm, idx_v, rows_v, sem):
        wid = jax.lax.axis_index("s") * SC_NC + jax.lax.axis_index("c")
        base = wid * rows_per_w
        for c in range(chunks):
            off = base + c * ch
            pltpu.sync_copy(rid_hbm.at[pl.ds(off, ch)], idx_v)
            pltpu.async_copy(x_hbm.at[idx_v], rows_v, sem).wait()
            pltpu.sync_copy(rows_v, out_hbm.at[pl.ds(off, ch)])

    return gather_k(x, rid)


def _grouped_body(teid_ref, xs_ref, wg_ref, wu_ref, wd_ref, rw_ref, og_ref):
    xs = xs_ref[...]
    g = jax.lax.dot_general(xs, wg_ref[0], (((1,), (1,)), ((), ())),
                            preferred_element_type=jnp.float32)
    u = jax.lax.dot_general(xs, wu_ref[0], (((1,), (1,)), ((), ())),
                            preferred_element_type=jnp.float32)
    h = (g * jax.nn.sigmoid(g)) * u
    o = jax.lax.dot_general(h, wd_ref[0], (((1,), (1,)), ((), ())),
                            preferred_element_type=jnp.float32)
    og_ref[...] = o * rw_ref[...]


def _grouped_mlp(xs, expert_gate, expert_up, expert_down, rw, teid, tm):
    pad_t, d = xs.shape
    ne, dff, _ = expert_gate.shape
    nt = pad_t // tm
    grid_spec = pltpu.PrefetchScalarGridSpec(
        num_scalar_prefetch=1,
        grid=(nt,),
        in_specs=[
            pl.BlockSpec((tm, d), lambda i, teid: (i, 0)),
            pl.BlockSpec((1, dff, d), lambda i, teid: (teid[i], 0, 0)),
            pl.BlockSpec((1, dff, d), lambda i, teid: (teid[i], 0, 0)),
            pl.BlockSpec((1, d, dff), lambda i, teid: (teid[i], 0, 0)),
            pl.BlockSpec((tm, 1), lambda i, teid: (i, 0)),
        ],
        out_specs=pl.BlockSpec((tm, d), lambda i, teid: (i, 0)),
    )
    return pl.pallas_call(
        _grouped_body,
        grid_spec=grid_spec,
        out_shape=jax.ShapeDtypeStruct((pad_t, d), jnp.float32),
    )(teid, xs, expert_gate, expert_up, expert_down, rw)


def _combine_body(og_ref, rid2_ref, shared_ref, out_ref):
    i = pl.program_id(0)

    @pl.when(i == 0)
    def _():
        out_ref[...] = shared_ref[...]

    t = out_ref.shape[0]
    tmc = og_ref.shape[0]
    onehot = (rid2_ref[...] == jax.lax.broadcasted_iota(jnp.int32, (tmc, t), 1)
              ).astype(jnp.float32)
    out_ref[...] += jax.lax.dot_general(onehot, og_ref[...],
                                        (((0,), (0,)), ((), ())),
                                        preferred_element_type=jnp.float32)


def _combine_tc(og, rid2, shared, tmc):
    pad_t, d = og.shape
    t = shared.shape[0]
    return pl.pallas_call(
        _combine_body,
        grid=(pad_t // tmc,),
        in_specs=[pl.BlockSpec((tmc, d), lambda i: (i, 0)),
                  pl.BlockSpec((tmc, 1), lambda i: (i, 0)),
                  pl.BlockSpec((t, d), lambda i: (0, 0))],
        out_specs=pl.BlockSpec((t, d), lambda i: (0, 0)),
        out_shape=jax.ShapeDtypeStruct((t, d), jnp.float32),
    )(og, rid2, shared)


def _combine_sc(og, pos, shared):
    t, d = shared.shape
    tok_per_w = t // SC_NW
    ch = 16
    chunks = tok_per_w // ch
    nvec = d // SC_LANES
    mesh = plsc.VectorSubcoreMesh(core_axis_name="c", subcore_axis_name="s",
                                  num_cores=SC_NC, num_subcores=SC_NS)

    @functools.partial(
        pl.kernel, mesh=mesh,
        out_type=jax.ShapeDtypeStruct((t, d), jnp.float32),
        scratch_types=[pltpu.VMEM((TOP_K * ch,), jnp.int32),
                       pltpu.VMEM((TOP_K * ch, d), jnp.float32),
                       pltpu.VMEM((ch, d), jnp.float32),
                       pltpu.SemaphoreType.DMA],
    )
    def combine_k(og_hbm, pos_hbm, sh_hbm, out_hbm, pos_v, rows_v, acc_v, sem):
        wid = jax.lax.axis_index("s") * SC_NC + jax.lax.axis_index("c")
        tbase = wid * tok_per_w
        for c in range(chunks):
            tok0 = tbase + c * ch
            pltpu.sync_copy(pos_hbm.at[pl.ds(TOP_K * tok0, TOP_K * ch)], pos_v)
            pltpu.async_copy(og_hbm.at[pos_v], rows_v, sem).wait()
            pltpu.sync_copy(sh_hbm.at[pl.ds(tok0, ch)], acc_v)

            def add_row(j, _):
                for l in range(nvec):
                    sl = pl.ds(l * SC_LANES, SC_LANES)
                    acc_v[j, sl] = (acc_v[j, sl] + rows_v[TOP_K * j, sl]
                                    + rows_v[TOP_K * j + 1, sl])
                return 0
            jax.lax.fori_loop(0, ch, add_row, 0)
            pltpu.sync_copy(acc_v, out_hbm.at[pl.ds(tok0, ch)])

    return combine_k(og, pos, shared)


def _shared_body(x_ref, wg_ref, wu_ref, wd_ref, out_ref):
    f = pl.program_id(0)

    @pl.when(f == 0)
    def _():
        out_ref[...] = jnp.zeros_like(out_ref)

    x = x_ref[...]
    g = jax.lax.dot_general(x, wg_ref[...], (((1,), (1,)), ((), ())),
                            preferred_element_type=jnp.float32)
    u = jax.lax.dot_general(x, wu_ref[...], (((1,), (1,)), ((), ())),
                            preferred_element_type=jnp.float32)
    h = (g * jax.nn.sigmoid(g)) * u
    out_ref[...] += jax.lax.dot_general(h, wd_ref[...], (((1,), (1,)), ((), ())),
                                        preferred_element_type=jnp.float32)


def _shared_mlp(x, wg, wu, wd, ffc=256):
    t, d = x.shape
    dffs = wg.shape[0]
    ffc = min(ffc, dffs)
    nf = dffs // ffc
    return pl.pallas_call(
        _shared_body,
        grid=(nf,),
        in_specs=[
            pl.BlockSpec((t, d), lambda f: (0, 0)),
            pl.BlockSpec((ffc, d), lambda f: (f, 0)),
            pl.BlockSpec((ffc, d), lambda f: (f, 0)),
            pl.BlockSpec((d, ffc), lambda f: (0, f)),
        ],
        out_specs=pl.BlockSpec((t, d), lambda f: (0, 0)),
        out_shape=jax.ShapeDtypeStruct((t, d), jnp.float32),
    )(x, wg, wu, wd)


def kernel(hidden_states, gate_weight, e_score_correction_bias, expert_gate,
           expert_up, expert_down, shared_gate, shared_up, shared_down):
    b, s, d = hidden_states.shape
    ne = gate_weight.shape[0]
    x = hidden_states.reshape(-1, d)
    t = x.shape[0]
    tm = min(TM, max(8, t // 8))
    pad_t = TOP_K * t + ne * tm
    nt = pad_t // tm
    tmc = min(512, pad_t)

    idx, w = _router(x, gate_weight, e_score_correction_bias)
    rid, rw, teid, pos = _dispatch(idx, w, ne, pad_t, nt, tm)
    rw2 = rw.reshape(pad_t, 1)
    use_sc = (t % SC_NW == 0) and (pad_t % (32 * SC_NW) == 0)
    if use_sc:
        xs = _gather_sc(x, rid, pad_t)
    else:
        xs = _gather_tc(x, rid.reshape(pad_t, 1), pad_t, tm)
    shared = _shared_mlp(x, shared_gate, shared_up, shared_down)
    og = _grouped_mlp(xs, expert_gate, expert_up, expert_down, rw2, teid, tm)
    if use_sc:
        out = _combine_sc(og, pos, shared)
    else:
        out = _combine_tc(og, rid.reshape(pad_t, 1), shared, tmc)
    return out.reshape(hidden_states.shape)


# speedup vs baseline: 1.7279x; 1.7279x over previous
import functools

import jax
import jax.numpy as jnp
from jax.experimental import pallas as pl
from jax.experimental.pallas import tpu as pltpu

TOP_K = 2
ROUTED_SCALING_FACTOR = 2.5


def _router_body(x_ref, gw_ref, bias_ref, combine_ref):
    x = x_ref[...]
    gw = gw_ref[...]
    logits = jax.lax.dot_general(x, gw, (((1,), (1,)), ((), ())),
                                 preferred_element_type=jnp.float32)
    scores = jax.nn.sigmoid(logits)
    s = scores + bias_ref[...]
    t, e = s.shape
    iota = jax.lax.broadcasted_iota(jnp.int32, (t, e), 1)
    big = jnp.asarray(e, jnp.int32)
    m1 = jnp.max(s, axis=1, keepdims=True)
    i1 = jnp.min(jnp.where(s == m1, iota, big), axis=1, keepdims=True)
    oh1 = (iota == i1)
    s2 = jnp.where(oh1, -jnp.inf, s)
    m2 = jnp.max(s2, axis=1, keepdims=True)
    i2 = jnp.min(jnp.where(s2 == m2, iota, big), axis=1, keepdims=True)
    oh2 = (iota == i2)
    w1 = jnp.sum(jnp.where(oh1, scores, 0.0), axis=1, keepdims=True)
    w2 = jnp.sum(jnp.where(oh2, scores, 0.0), axis=1, keepdims=True)
    scale = ROUTED_SCALING_FACTOR / (w1 + w2 + 1e-20)
    combine_ref[...] = (jnp.where(oh1, w1, 0.0) + jnp.where(oh2, w2, 0.0)) * scale


def _router(x, gate_weight, bias):
    t, d = x.shape
    e = gate_weight.shape[0]
    return pl.pallas_call(
        _router_body,
        out_shape=jax.ShapeDtypeStruct((t, e), jnp.float32),
    )(x, gate_weight, bias.reshape(1, e))


def _moe_dense_body(x_ref, comb_ref, wg_ref, wu_ref, wd_ref, out_ref):
    e = pl.program_id(0)
    f = pl.program_id(1)

    @pl.when(jnp.logical_and(e == 0, f == 0))
    def _():
        out_ref[...] = jnp.zeros_like(out_ref)

    x = x_ref[...]
    g = jax.lax.dot_general(x, wg_ref[0], (((1,), (1,)), ((), ())),
                            preferred_element_type=jnp.float32)
    u = jax.lax.dot_general(x, wu_ref[0], (((1,), (1,)), ((), ())),
                            preferred_element_type=jnp.float32)
    h = ((g * jax.nn.sigmoid(g)) * u).astype(wd_ref.dtype)
    o = jax.lax.dot_general(h, wd_ref[0], (((1,), (1,)), ((), ())),
                            preferred_element_type=jnp.float32)
    comb = comb_ref[...]
    lane = jax.lax.broadcasted_iota(jnp.int32, comb.shape, 1)
    wcol = jnp.sum(jnp.where(lane == e, comb, 0.0), axis=1, keepdims=True)
    out_ref[...] += o * wcol


def _moe_dense(x, combine, expert_gate, expert_up, expert_down, ffc=256):
    t, d = x.shape
    ne, dff, _ = expert_gate.shape
    ffc = min(ffc, dff)
    nf = dff // ffc
    return pl.pallas_call(
        _moe_dense_body,
        grid=(ne, nf),
        in_specs=[
            pl.BlockSpec((t, d), lambda e, f: (0, 0)),
            pl.BlockSpec((t, ne), lambda e, f: (0, 0)),
            pl.BlockSpec((1, ffc, d), lambda e, f: (e, f, 0)),
            pl.BlockSpec((1, ffc, d), lambda e, f: (e, f, 0)),
            pl.BlockSpec((1, d, ffc), lambda e, f: (e, 0, f)),
        ],
        out_specs=pl.BlockSpec((t, d), lambda e, f: (0, 0)),
        out_shape=jax.ShapeDtypeStruct((t, d), jnp.float32),
    )(x, combine, expert_gate, expert_up, expert_down)


def _shared_body(x_ref, wg_ref, wu_ref, wd_ref, out_ref):
    f = pl.program_id(0)

    @pl.when(f == 0)
    def _():
        out_ref[...] = jnp.zeros_like(out_ref)

    x = x_ref[...]
    g = jax.lax.dot_general(x, wg_ref[...], (((1,), (1,)), ((), ())),
                            preferred_element_type=jnp.float32)
    u = jax.lax.dot_general(x, wu_ref[...], (((1,), (1,)), ((), ())),
                            preferred_element_type=jnp.float32)
    h = ((g * jax.nn.sigmoid(g)) * u).astype(wd_ref.dtype)
    out_ref[...] += jax.lax.dot_general(h, wd_ref[...], (((1,), (1,)), ((), ())),
                                        preferred_element_type=jnp.float32)


def _shared_mlp(x, wg, wu, wd, ffc=256):
    t, d = x.shape
    dffs = wg.shape[0]
    ffc = min(ffc, dffs)
    nf = dffs // ffc
    return pl.pallas_call(
        _shared_body,
        grid=(nf,),
        in_specs=[
            pl.BlockSpec((t, d), lambda f: (0, 0)),
            pl.BlockSpec((ffc, d), lambda f: (f, 0)),
            pl.BlockSpec((ffc, d), lambda f: (f, 0)),
            pl.BlockSpec((d, ffc), lambda f: (0, f)),
        ],
        out_specs=pl.BlockSpec((t, d), lambda f: (0, 0)),
        out_shape=jax.ShapeDtypeStruct((t, d), jnp.float32),
    )(x, wg, wu, wd)


def kernel(hidden_states, gate_weight, e_score_correction_bias, expert_gate,
           expert_up, expert_down, shared_gate, shared_up, shared_down):
    b, s, d = hidden_states.shape
    x = hidden_states.reshape(-1, d)
    combine = _router(x, gate_weight, e_score_correction_bias)
    routed = _moe_dense(x, combine, expert_gate, expert_up, expert_down)
    shared = _shared_mlp(x, shared_gate, shared_up, shared_down)
    return (routed + shared).reshape(hidden_states.shape)
